# baseline (speedup 1.0000x reference)
import functools

import jax
import jax.numpy as jnp
from jax import lax
from jax.experimental import pallas as pl
from jax.experimental.pallas import tpu as pltpu
from jax.experimental.pallas import tpu_sc as plsc

N_PAD = 51200
TILE_ROWS = N_PAD // 16
CHUNK_ROWS = 32
N_CHUNKS = TILE_ROWS // CHUNK_ROWS
DEG = 16
F = 64
FH = 32
BLK = 2048
_GDN = lax.GatherDimensionNumbers(
    offset_dims=(), collapsed_slice_dims=(0,), start_index_map=(0,))


def _lane(v, j):
    idx = jnp.full((16, 1), j, jnp.int32)
    return lax.gather(v, idx, _GDN, (1,),
                      mode=lax.GatherScatterMode.PROMISE_IN_BOUNDS)


def _prep_body(x_ref, w_ref, att_ref, xl_ref, a2_ref):
    x = x_ref[...]
    xl = jnp.dot(x, w_ref[...], preferred_element_type=jnp.float32)
    xl_ref[...] = xl
    a2_ref[...] = jnp.dot(xl, att_ref[...].T, preferred_element_type=jnp.float32)


def _tc_prep(xp, w, att2):
    grid = (N_PAD // BLK,)
    return pl.pallas_call(
        _prep_body,
        grid=grid,
        in_specs=[
            pl.BlockSpec((BLK, F), lambda i: (i, 0)),
            pl.BlockSpec((F, F), lambda i: (0, 0)),
            pl.BlockSpec((2, F), lambda i: (0, 0)),
        ],
        out_specs=[
            pl.BlockSpec((BLK, F), lambda i: (i, 0)),
            pl.BlockSpec((BLK, 2), lambda i: (i, 0)),
        ],
        out_shape=[
            jax.ShapeDtypeStruct((N_PAD, F), jnp.float32),
            jax.ShapeDtypeStruct((N_PAD, 2), jnp.float32),
        ],
    )(xp, w, att2)


FQ = 16


def _edge_body(e_hbm, adst_hbm, asrc_hbm, xl_hbm, ash_hbm,
               nums_hbm, den_hbm,
               adst_t, evb, xlb, asb, msg, idxb, wbuf, avb,
               sem_in, sem_sc, acc, dacc):
    c = lax.axis_index("c")
    s = lax.axis_index("s")
    base_row = s * TILE_ROWS

    pltpu.sync_copy(adst_hbm, adst_t)
    pltpu.sync_copy(ash_hbm, avb)
    avec = avb[...]
    stream_denom = jnp.logical_or(
        jnp.logical_and(s < 8, c == 0), jnp.logical_and(s >= 8, c == 1))
    z16 = jnp.zeros((16,), jnp.float32)

    for phase in range(2):
        grp = 2 * phase + c

        def in_copies(ci, b):
            r0 = base_row + ci * CHUNK_ROWS
            return (
                pltpu.make_async_copy(
                    e_hbm.at[pl.ds(r0, CHUNK_ROWS), :], evb.at[b],
                    sem_in.at[b]),
                pltpu.make_async_copy(
                    xl_hbm.at[pl.ds(r0, CHUNK_ROWS), pl.ds(grp * FQ, FQ)],
                    xlb.at[b], sem_in.at[b]),
                pltpu.make_async_copy(
                    asrc_hbm.at[pl.ds(r0, CHUNK_ROWS)], asb.at[b],
                    sem_in.at[b]),
            )

        for i in range(128):
            msg[0, i, :] = z16
        for i in range(8):
            wbuf[0, 0, pl.ds(i * 16, 16)] = z16
        for k in range(TILE_ROWS // 128):
            pltpu.sync_copy(msg.at[0, pl.ds(0, 128), :],
                            acc.at[pl.ds(base_row + k * 128, 128), :])
            if phase == 0:
                pltpu.sync_copy(wbuf.at[0, 0],
                                dacc.at[pl.ds(base_row + k * 128, 128)])
        plsc.subcore_barrier()

        def wait_sc(b):
            for g in range(CHUNK_ROWS // 8):
                pltpu.make_async_copy(
                    msg.at[b, pl.ds(g * 128, 128), :],
                    acc.at[idxb.at[b, g]], sem_sc.at[b]).wait()
                if phase == 0:
                    @pl.when(stream_denom)
                    def _():
                        pltpu.make_async_copy(
                            wbuf.at[b, g], dacc.at[idxb.at[b, g]],
                            sem_sc.at[b]).wait()

        for cp in in_copies(0, 0):
            cp.start()

        def do_chunk(ci, b, first):
            for cp in in_copies(ci, b):
                cp.wait()

            @pl.when(ci + 1 < N_CHUNKS)
            def _():
                for cp in in_copies(ci + 1, 1 - b):
                    cp.start()

            @pl.when(jnp.logical_not(first))
            def _():
                wait_sc(b)

            for r in range(CHUNK_ROWS):
                g, p = r // 8, (r % 8) * 16
                avs = asb[b, pl.ds((r // 16) * 16, 16)]
                idx16 = evb[b, r, :]
                ad = plsc.load_gather(adst_t, [idx16])
                alpha = _lane(avs, r % 16) + ad
                lr = jnp.where(alpha >= 0.0, alpha, alpha * 0.2)
                w = jnp.exp(lr - avec)
                wbuf[b, g, pl.ds(p, 16)] = w
                idxb[b, g, pl.ds(p, 16)] = idx16
                xr = xlb[b, r, :]
                for j in range(16):
                    msg[b, r * 16 + j, :] = _lane(w, j) * xr
            for g in range(CHUNK_ROWS // 8):
                pltpu.async_copy(msg.at[b, pl.ds(g * 128, 128), :],
                                 acc.at[idxb.at[b, g]], sem_sc.at[b],
                                 add=True)
                if phase == 0:
                    @pl.when(stream_denom)
                    def _():
                        pltpu.async_copy(wbuf.at[b, g],
                                         dacc.at[idxb.at[b, g]],
                                         sem_sc.at[b], add=True)

        def pair(i2, carry):
            first = i2 == 0
            do_chunk(2 * i2, 0, first)
            do_chunk(2 * i2 + 1, 1, first)
            return carry

        lax.fori_loop(0, N_CHUNKS // 2, pair, 0)
        wait_sc(0)
        wait_sc(1)
        plsc.subcore_barrier()

        pltpu.sync_copy(
            acc.at[pl.ds(base_row, TILE_ROWS), :],
            nums_hbm.at[pl.ds(base_row, TILE_ROWS), pl.ds(grp * FQ, FQ)])
        if phase == 0:
            pltpu.sync_copy(dacc.at[pl.ds(base_row, TILE_ROWS)],
                            den_hbm.at[c, pl.ds(base_row, TILE_ROWS)])
        plsc.subcore_barrier()


def _sc_edge(ep, adst, asrc, xl, ash):
    mesh = plsc.VectorSubcoreMesh(core_axis_name="c", subcore_axis_name="s")
    f = functools.partial(
        pl.kernel,
        out_type=(
            jax.ShapeDtypeStruct((N_PAD, F), jnp.float32),
            jax.ShapeDtypeStruct((2, N_PAD), jnp.float32),
        ),
        mesh=mesh,
        scratch_types=[
            pltpu.VMEM((N_PAD,), jnp.float32),
            pltpu.VMEM((2, CHUNK_ROWS, DEG), jnp.int32),
            pltpu.VMEM((2, CHUNK_ROWS, FQ), jnp.float32),
            pltpu.VMEM((2, CHUNK_ROWS), jnp.float32),
            pltpu.VMEM((2, 16 * CHUNK_ROWS, FQ), jnp.float32),
            pltpu.VMEM((2, CHUNK_ROWS // 8, 128), jnp.int32),
            pltpu.VMEM((2, CHUNK_ROWS // 8, 128), jnp.float32),
            pltpu.VMEM((16,), jnp.float32),
            pltpu.SemaphoreType.DMA((2,)),
            pltpu.SemaphoreType.DMA((2,)),
            pltpu.VMEM_SHARED((N_PAD, FQ), jnp.float32),
            pltpu.VMEM_SHARED((N_PAD,), jnp.float32),
        ],
        compiler_params=pltpu.CompilerParams(
            needs_layout_passes=False, use_tc_tiling_on_sc=False),
    )(_edge_body)
    return f(ep, adst, asrc, xl, ash)


def _epi_body(num_ref, xl_ref, a2_ref, den_ref, ash_ref, bias_ref, g_ref):
    num = num_ref[...]
    xl = xl_ref[...]
    asrc = a2_ref[:, 0:1]
    adst = a2_ref[:, 1:2]
    den = den_ref[...]
    a = ash_ref[0, 0]
    al = asrc + adst
    lr = jnp.where(al >= 0.0, al, al * 0.2)
    ws = jnp.exp(lr - a)
    g = (num + ws * xl) / (den + ws + 1e-16)
    gb = g + bias_ref[...]
    nrm = jnp.sqrt(jnp.sum(gb * gb, axis=1, keepdims=True))
    g_ref[...] = gb / jnp.maximum(nrm, 1e-12)


def _tc_epilogue(num, xl, a2, den, ash, bias2d):
    grid = (N_PAD // BLK,)
    return pl.pallas_call(
        _epi_body,
        grid=grid,
        in_specs=[
            pl.BlockSpec((BLK, F), lambda i: (i, 0)),
            pl.BlockSpec((BLK, F), lambda i: (i, 0)),
            pl.BlockSpec((BLK, 2), lambda i: (i, 0)),
            pl.BlockSpec((BLK, 1), lambda i: (i, 0)),
            pl.BlockSpec((1, 1), lambda i: (0, 0)),
            pl.BlockSpec((1, F), lambda i: (0, 0)),
        ],
        out_specs=pl.BlockSpec((BLK, F), lambda i: (i, 0)),
        out_shape=jax.ShapeDtypeStruct((N_PAD, F), jnp.float32),
    )(num, xl, a2, den, ash, bias2d)


def _gather_body(user_hbm, pos_hbm, neg_hbm, ae_hbm, g_hbm, out_hbm,
                 idxv, rows, sem):
    c = lax.axis_index("c")
    s = lax.axis_index("s")
    wid = s * 2 + c
    base = wid * 128
    for k, idx_hbm in enumerate((user_hbm, pos_hbm, neg_hbm)):
        pltpu.sync_copy(idx_hbm.at[pl.ds(base, 128)], idxv)
        pltpu.async_copy(ae_hbm.at[idxv], rows, sem).wait()
        pltpu.sync_copy(rows, out_hbm.at[2 * k, pl.ds(base, 128), :])
        pltpu.async_copy(g_hbm.at[idxv], rows, sem).wait()
        pltpu.sync_copy(rows, out_hbm.at[2 * k + 1, pl.ds(base, 128), :])


def _sc_gather(user, pos_item, neg_item, all_embed, g):
    mesh = plsc.VectorSubcoreMesh(core_axis_name="c", subcore_axis_name="s")
    f = functools.partial(
        pl.kernel,
        out_type=jax.ShapeDtypeStruct((6, 4096, F), jnp.float32),
        mesh=mesh,
        scratch_types=[
            pltpu.VMEM((128,), jnp.int32),
            pltpu.VMEM((128, F), jnp.float32),
            pltpu.SemaphoreType.DMA,
        ],
        compiler_params=pltpu.CompilerParams(
            needs_layout_passes=False, use_tc_tiling_on_sc=False),
    )(_gather_body)
    return f(user, pos_item, neg_item, all_embed, g)


def _loss_body(x_ref, out_ref):
    x = x_ref[...]
    au, gu, ap, gp, an, gn = x[0], x[1], x[2], x[3], x[4], x[5]
    ps = jnp.sum(au * ap + gu * gp, axis=1)
    ns = jnp.sum(au * an + gu * gn, axis=1)
    d = ps - ns
    bpr = jnp.mean(jnp.maximum(-d, 0.0) + jnp.log(1.0 + jnp.exp(-jnp.abs(d))))
    reg = 1e-5 * 0.5 * jnp.sum(x * x)
    out_ref[...] = jnp.reshape(bpr + reg, (1, 1))


def _tc_loss(gath):
    return pl.pallas_call(
        _loss_body,
        out_shape=jax.ShapeDtypeStruct((1, 1), jnp.float32),
    )(gath)


def kernel(user, pos_item, neg_item, edges_matrix, all_embed,
           entity_embedding, W, att_src, att_dst, bias):
    n = entity_embedding.shape[0]
    pad = N_PAD - n
    xp = jnp.concatenate(
        [entity_embedding, jnp.zeros((pad, F), jnp.float32)], axis=0)
    ep = jnp.concatenate(
        [edges_matrix, jnp.full((pad, DEG), N_PAD - 1, jnp.int32)], axis=0)
    att2 = jnp.stack([att_src, att_dst], axis=0)

    xl, a2 = _tc_prep(xp, W, att2)
    asrc = a2[:, 0]
    adst = a2[:, 1]
    m = jnp.max(asrc) + jnp.max(adst)
    a_shift = jnp.where(m >= 0.0, m, 0.2 * m)
    ash = jnp.full((16,), 1.0, jnp.float32) * a_shift

    num, den2 = _sc_edge(ep, adst, asrc, xl, ash)
    den = (den2[0] + den2[1]).reshape(N_PAD, 1)
    g = _tc_epilogue(num, xl, a2, den, a_shift.reshape(1, 1),
                     bias.reshape(1, F))

    gath = _sc_gather(user, pos_item, neg_item, all_embed, g)
    loss = _tc_loss(gath)
    return loss[0, 0]


# speedup vs baseline: 1.1322x; 1.1322x over previous
import functools

import jax
import jax.numpy as jnp
from jax import lax
from jax.experimental import pallas as pl
from jax.experimental.pallas import tpu as pltpu
from jax.experimental.pallas import tpu_sc as plsc

N_PAD = 51200
TILE_ROWS = N_PAD // 16
CHUNK_ROWS = 16
N_CHUNKS = TILE_ROWS // CHUNK_ROWS
DEG = 16
F = 64
FH = 32
BLK = 2048
_GDN = lax.GatherDimensionNumbers(
    offset_dims=(), collapsed_slice_dims=(0,), start_index_map=(0,))


def _lane(v, j):
    idx = jnp.full((16, 1), j, jnp.int32)
    return lax.gather(v, idx, _GDN, (1,),
                      mode=lax.GatherScatterMode.PROMISE_IN_BOUNDS)


def _prep_body(x_ref, w_ref, att_ref, xl_ref, a2_ref):
    x = x_ref[...]
    xl = jnp.dot(x, w_ref[...], preferred_element_type=jnp.float32)
    xl_ref[...] = xl
    a2_ref[...] = jnp.dot(xl, att_ref[...].T, preferred_element_type=jnp.float32)


def _tc_prep(xp, w, att2):
    grid = (N_PAD // BLK,)
    return pl.pallas_call(
        _prep_body,
        grid=grid,
        in_specs=[
            pl.BlockSpec((BLK, F), lambda i: (i, 0)),
            pl.BlockSpec((F, F), lambda i: (0, 0)),
            pl.BlockSpec((2, F), lambda i: (0, 0)),
        ],
        out_specs=[
            pl.BlockSpec((BLK, F), lambda i: (i, 0)),
            pl.BlockSpec((BLK, 2), lambda i: (i, 0)),
        ],
        out_shape=[
            jax.ShapeDtypeStruct((N_PAD, F), jnp.float32),
            jax.ShapeDtypeStruct((N_PAD, 2), jnp.float32),
        ],
    )(xp, w, att2)


FQ = 16


def _edge_body(e_hbm, adst_hbm, asrc_hbm, xl_hbm, ash_hbm,
               nums_hbm, den_hbm,
               adst_t, evb, xlb, asb, msg, idxb, wbuf, avb,
               sem_in, sem_sc, acc, dacc):
    c = lax.axis_index("c")
    s = lax.axis_index("s")
    base_row = s * TILE_ROWS

    pltpu.sync_copy(adst_hbm, adst_t)
    pltpu.sync_copy(ash_hbm, avb)
    avec = avb[...]
    stream_denom = jnp.logical_or(
        jnp.logical_and(s < 8, c == 0), jnp.logical_and(s >= 8, c == 1))
    z16 = jnp.zeros((16,), jnp.float32)

    for phase in range(2):
        grp = 2 * phase + c

        def in_copies(ci, b):
            r0 = base_row + ci * CHUNK_ROWS
            return (
                pltpu.make_async_copy(
                    e_hbm.at[pl.ds(r0, CHUNK_ROWS), :], evb.at[b],
                    sem_in.at[b]),
                pltpu.make_async_copy(
                    xl_hbm.at[pl.ds(r0, CHUNK_ROWS), pl.ds(grp * FQ, FQ)],
                    xlb.at[b], sem_in.at[b]),
                pltpu.make_async_copy(
                    asrc_hbm.at[pl.ds(r0, CHUNK_ROWS)], asb.at[b],
                    sem_in.at[b]),
            )

        for i in range(128):
            msg[0, i, :] = z16
        for i in range(8):
            wbuf[0, 0, pl.ds(i * 16, 16)] = z16
        for k in range(TILE_ROWS // 128):
            pltpu.sync_copy(msg.at[0, pl.ds(0, 128), :],
                            acc.at[pl.ds(base_row + k * 128, 128), :])
            if phase == 0:
                pltpu.sync_copy(wbuf.at[0, 0],
                                dacc.at[pl.ds(base_row + k * 128, 128)])
        plsc.subcore_barrier()

        def wait_sc(b):
            for g in range(CHUNK_ROWS // 8):
                pltpu.make_async_copy(
                    msg.at[b, pl.ds(g * 128, 128), :],
                    acc.at[idxb.at[b, g]], sem_sc.at[b]).wait()
                if phase == 0:
                    @pl.when(stream_denom)
                    def _():
                        pltpu.make_async_copy(
                            wbuf.at[b, g], dacc.at[idxb.at[b, g]],
                            sem_sc.at[b]).wait()

        for cp in in_copies(0, 0):
            cp.start()

        def do_chunk(ci, b, first):
            for cp in in_copies(ci, b):
                cp.wait()

            @pl.when(ci + 1 < N_CHUNKS)
            def _():
                for cp in in_copies(ci + 1, 1 - b):
                    cp.start()

            @pl.when(jnp.logical_not(first))
            def _():
                wait_sc(b)

            for r in range(CHUNK_ROWS):
                g, p = r // 8, (r % 8) * 16
                avs = asb[b, pl.ds((r // 16) * 16, 16)]
                idx16 = evb[b, r, :]
                ad = plsc.load_gather(adst_t, [idx16])
                alpha = _lane(avs, r % 16) + ad
                lr = jnp.where(alpha >= 0.0, alpha, alpha * 0.2)
                w = jnp.exp(lr - avec)
                wbuf[b, g, pl.ds(p, 16)] = w
                idxb[b, g, pl.ds(p, 16)] = idx16
                xr = xlb[b, r, :]
                for j in range(16):
                    msg[b, r * 16 + j, :] = _lane(w, j) * xr
            for g in range(CHUNK_ROWS // 8):
                pltpu.async_copy(msg.at[b, pl.ds(g * 128, 128), :],
                                 acc.at[idxb.at[b, g]], sem_sc.at[b],
                                 add=True)
                if phase == 0:
                    @pl.when(stream_denom)
                    def _():
                        pltpu.async_copy(wbuf.at[b, g],
                                         dacc.at[idxb.at[b, g]],
                                         sem_sc.at[b], add=True)

        def pair(i2, carry):
            first = i2 == 0
            do_chunk(2 * i2, 0, first)
            do_chunk(2 * i2 + 1, 1, first)
            return carry

        lax.fori_loop(0, N_CHUNKS // 2, pair, 0)
        wait_sc(0)
        wait_sc(1)
        plsc.subcore_barrier()

        pltpu.sync_copy(
            acc.at[pl.ds(base_row, TILE_ROWS), :],
            nums_hbm.at[pl.ds(base_row, TILE_ROWS), pl.ds(grp * FQ, FQ)])
        if phase == 0:
            pltpu.sync_copy(dacc.at[pl.ds(base_row, TILE_ROWS)],
                            den_hbm.at[c, pl.ds(base_row, TILE_ROWS)])
        plsc.subcore_barrier()


def _sc_edge(ep, adst, asrc, xl, ash):
    mesh = plsc.VectorSubcoreMesh(core_axis_name="c", subcore_axis_name="s")
    f = functools.partial(
        pl.kernel,
        out_type=(
            jax.ShapeDtypeStruct((N_PAD, F), jnp.float32),
            jax.ShapeDtypeStruct((2, N_PAD), jnp.float32),
        ),
        mesh=mesh,
        scratch_types=[
            pltpu.VMEM((N_PAD,), jnp.float32),
            pltpu.VMEM((2, CHUNK_ROWS, DEG), jnp.int32),
            pltpu.VMEM((2, CHUNK_ROWS, FQ), jnp.float32),
            pltpu.VMEM((2, CHUNK_ROWS), jnp.float32),
            pltpu.VMEM((2, 16 * CHUNK_ROWS, FQ), jnp.float32),
            pltpu.VMEM((2, CHUNK_ROWS // 8, 128), jnp.int32),
            pltpu.VMEM((2, CHUNK_ROWS // 8, 128), jnp.float32),
            pltpu.VMEM((16,), jnp.float32),
            pltpu.SemaphoreType.DMA((2,)),
            pltpu.SemaphoreType.DMA((2,)),
            pltpu.VMEM_SHARED((N_PAD, FQ), jnp.float32),
            pltpu.VMEM_SHARED((N_PAD,), jnp.float32),
        ],
        compiler_params=pltpu.CompilerParams(
            needs_layout_passes=False, use_tc_tiling_on_sc=False),
    )(_edge_body)
    return f(ep, adst, asrc, xl, ash)


def _epi_body(num_ref, xl_ref, a2_ref, den_ref, ash_ref, bias_ref, g_ref):
    num = num_ref[...]
    xl = xl_ref[...]
    asrc = a2_ref[:, 0:1]
    adst = a2_ref[:, 1:2]
    den = den_ref[...]
    a = ash_ref[0, 0]
    al = asrc + adst
    lr = jnp.where(al >= 0.0, al, al * 0.2)
    ws = jnp.exp(lr - a)
    g = (num + ws * xl) / (den + ws + 1e-16)
    gb = g + bias_ref[...]
    nrm = jnp.sqrt(jnp.sum(gb * gb, axis=1, keepdims=True))
    g_ref[...] = gb / jnp.maximum(nrm, 1e-12)


def _tc_epilogue(num, xl, a2, den, ash, bias2d):
    grid = (N_PAD // BLK,)
    return pl.pallas_call(
        _epi_body,
        grid=grid,
        in_specs=[
            pl.BlockSpec((BLK, F), lambda i: (i, 0)),
            pl.BlockSpec((BLK, F), lambda i: (i, 0)),
            pl.BlockSpec((BLK, 2), lambda i: (i, 0)),
            pl.BlockSpec((BLK, 1), lambda i: (i, 0)),
            pl.BlockSpec((1, 1), lambda i: (0, 0)),
            pl.BlockSpec((1, F), lambda i: (0, 0)),
        ],
        out_specs=pl.BlockSpec((BLK, F), lambda i: (i, 0)),
        out_shape=jax.ShapeDtypeStruct((N_PAD, F), jnp.float32),
    )(num, xl, a2, den, ash, bias2d)


def _gather_body(user_hbm, pos_hbm, neg_hbm, ae_hbm, g_hbm, out_hbm,
                 idxv, rows, sem):
    c = lax.axis_index("c")
    s = lax.axis_index("s")
    wid = s * 2 + c
    base = wid * 128
    for k, idx_hbm in enumerate((user_hbm, pos_hbm, neg_hbm)):
        pltpu.sync_copy(idx_hbm.at[pl.ds(base, 128)], idxv)
        pltpu.async_copy(ae_hbm.at[idxv], rows, sem).wait()
        pltpu.sync_copy(rows, out_hbm.at[2 * k, pl.ds(base, 128), :])
        pltpu.async_copy(g_hbm.at[idxv], rows, sem).wait()
        pltpu.sync_copy(rows, out_hbm.at[2 * k + 1, pl.ds(base, 128), :])


def _sc_gather(user, pos_item, neg_item, all_embed, g):
    mesh = plsc.VectorSubcoreMesh(core_axis_name="c", subcore_axis_name="s")
    f = functools.partial(
        pl.kernel,
        out_type=jax.ShapeDtypeStruct((6, 4096, F), jnp.float32),
        mesh=mesh,
        scratch_types=[
            pltpu.VMEM((128,), jnp.int32),
            pltpu.VMEM((128, F), jnp.float32),
            pltpu.SemaphoreType.DMA,
        ],
        compiler_params=pltpu.CompilerParams(
            needs_layout_passes=False, use_tc_tiling_on_sc=False),
    )(_gather_body)
    return f(user, pos_item, neg_item, all_embed, g)


def _loss_body(x_ref, out_ref):
    x = x_ref[...]
    au, gu, ap, gp, an, gn = x[0], x[1], x[2], x[3], x[4], x[5]
    ps = jnp.sum(au * ap + gu * gp, axis=1)
    ns = jnp.sum(au * an + gu * gn, axis=1)
    d = ps - ns
    bpr = jnp.mean(jnp.maximum(-d, 0.0) + jnp.log(1.0 + jnp.exp(-jnp.abs(d))))
    reg = 1e-5 * 0.5 * jnp.sum(x * x)
    out_ref[...] = jnp.reshape(bpr + reg, (1, 1))


def _tc_loss(gath):
    return pl.pallas_call(
        _loss_body,
        out_shape=jax.ShapeDtypeStruct((1, 1), jnp.float32),
    )(gath)


def kernel(user, pos_item, neg_item, edges_matrix, all_embed,
           entity_embedding, W, att_src, att_dst, bias):
    n = entity_embedding.shape[0]
    pad = N_PAD - n
    xp = jnp.concatenate(
        [entity_embedding, jnp.zeros((pad, F), jnp.float32)], axis=0)
    ep = jnp.concatenate(
        [edges_matrix, jnp.full((pad, DEG), N_PAD - 1, jnp.int32)], axis=0)
    att2 = jnp.stack([att_src, att_dst], axis=0)

    xl, a2 = _tc_prep(xp, W, att2)
    asrc = a2[:, 0]
    adst = a2[:, 1]
    m = jnp.max(asrc) + jnp.max(adst)
    a_shift = jnp.where(m >= 0.0, m, 0.2 * m)
    ash = jnp.full((16,), 1.0, jnp.float32) * a_shift

    num, den2 = _sc_edge(ep, adst, asrc, xl, ash)
    den = (den2[0] + den2[1]).reshape(N_PAD, 1)
    g = _tc_epilogue(num, xl, a2, den, a_shift.reshape(1, 1),
                     bias.reshape(1, F))

    gath = _sc_gather(user, pos_item, neg_item, all_embed, g)
    loss = _tc_loss(gath)
    return loss[0, 0]
